# Initial kernel scaffold; baseline (speedup 1.0000x reference)
import jax
import jax.numpy as jnp
from jax.experimental import pallas as pl


def kernel(x, edge_index, edge_weight, batch, W1, b1, W2, b2, W3, b3, Wc0, bc0, Wc1, bc1, Wc2, bc2):
    raise NotImplementedError("write your pallas kernel here")


# speedup vs baseline: 5.4027x; 5.4027x over previous
import functools

import jax
import jax.numpy as jnp
from jax import lax
from jax.experimental import pallas as pl
from jax.experimental.pallas import tpu as pltpu
from jax.experimental.pallas import tpu_sc as plsc

F32 = jnp.float32
I32 = jnp.int32

_NP = 10240
_H = 128
_G = 64
_C = 128
_NT = 16
_NC = 2
_RPT = _NP // _NT


def _mesh():
    return plsc.VectorSubcoreMesh(core_axis_name="c", subcore_axis_name="s")


def _fill_rows(ref, nrows, value):
    z = jnp.full((16,), value, F32)
    ncol = ref.shape[1] // 16

    def body(i, _):
        for j in range(ncol):
            ref[i, pl.ds(j * 16, 16)] = z
        return 0

    lax.fori_loop(0, nrows, body, 0)


def _deg_call(dst, w, ept32):
    kfn = functools.partial(
        pl.kernel,
        mesh=_mesh(),
        out_type=[
            jax.ShapeDtypeStruct((_NP, _H), F32),
            jax.ShapeDtypeStruct((_NP, _H), F32),
        ],
        scratch_types=[
            pltpu.VMEM_SHARED((_NP, _H), F32),
            pltpu.VMEM((_C, _H), F32),
            pltpu.VMEM((_C,), I32),
            pltpu.VMEM((_C,), F32),
        ],
    )

    @kfn
    def k(dst_h, w_h, deg0_h, deg1_h, deg_s, rows_v, idx_v, w_v):
        c = lax.axis_index("c")
        s = lax.axis_index("s")
        r0 = s * _RPT
        wid = c * _NT + s

        _fill_rows(rows_v, _C, 0.0)
        for t in range(_RPT // _C):
            pltpu.sync_copy(rows_v, deg_s.at[pl.ds(r0 + t * _C, _C)])
        plsc.subcore_barrier()

        def deg_chunk(kk, _):
            base = wid * ept32 + kk * _C
            pltpu.sync_copy(dst_h.at[pl.ds(base, _C)], idx_v)
            pltpu.sync_copy(w_h.at[pl.ds(base, _C)], w_v)

            def fill(g, __):
                w16 = w_v[pl.ds(g * 16, 16)]
                for l in range(16):
                    nv = jnp.full((16,), w16[l], F32)
                    for j in range(_H // 16):
                        rows_v[g * 16 + l, pl.ds(j * 16, 16)] = nv
                return 0

            lax.fori_loop(0, _C // 16, fill, 0)
            pltpu.sync_copy(rows_v, deg_s.at[idx_v], add=True)
            return 0

        lax.fori_loop(0, ept32 // _C, deg_chunk, 0)
        plsc.subcore_barrier()

        @pl.when(c == 0)
        def _wb0():
            pltpu.sync_copy(deg_s.at[pl.ds(r0, _RPT)],
                            deg0_h.at[pl.ds(r0, _RPT)])

        @pl.when(c == 1)
        def _wb1():
            pltpu.sync_copy(deg_s.at[pl.ds(r0, _RPT)],
                            deg1_h.at[pl.ds(r0, _RPT)])

    return k(dst, w)


def _edge_loop(hw_ref, src_h, dst_h, w_h, acc_s, rows_v, src_v, dst_v,
               w_v, sem, s, ept16):
    def chunk(kk, _):
        base = s * ept16 + kk * _C
        pltpu.sync_copy(src_h.at[pl.ds(base, _C)], src_v)
        pltpu.sync_copy(dst_h.at[pl.ds(base, _C)], dst_v)
        pltpu.sync_copy(w_h.at[pl.ds(base, _C)], w_v)
        pltpu.async_copy(hw_ref.at[src_v], rows_v, sem).wait()

        def scale(g, __):
            w16 = w_v[pl.ds(g * 16, 16)]
            for l in range(16):
                e = g * 16 + l
                nv = jnp.full((16,), w16[l], F32)
                for j in range(_H // 16):
                    sl = pl.ds(j * 16, 16)
                    rows_v[e, sl] = rows_v[e, sl] * nv
            return 0

        lax.fori_loop(0, _C // 16, scale, 0)
        pltpu.sync_copy(rows_v, acc_s.at[dst_v], add=True)
        return 0

    lax.fori_loop(0, ept16 // _C, chunk, 0)


def _spmm_call(hw0, hw1, src, dst, w, ept16):
    kfn = functools.partial(
        pl.kernel,
        mesh=_mesh(),
        out_type=[
            jax.ShapeDtypeStruct((_NP, _H), F32),
            jax.ShapeDtypeStruct((_NP, _H), F32),
        ],
        scratch_types=[
            pltpu.VMEM_SHARED((_NP, _H), F32),
            pltpu.VMEM((_C, _H), F32),
            pltpu.VMEM((_C,), I32),
            pltpu.VMEM((_C,), I32),
            pltpu.VMEM((_C,), F32),
            pltpu.SemaphoreType.DMA,
        ],
    )

    @kfn
    def k(hw0_h, hw1_h, src_h, dst_h, w_h, r0_h, r1_h, acc_s, rows_v,
          src_v, dst_v, w_v, sem):
        c = lax.axis_index("c")
        s = lax.axis_index("s")
        r0 = s * _RPT

        _fill_rows(rows_v, _C, 0.0)
        for t in range(_RPT // _C):
            pltpu.sync_copy(rows_v, acc_s.at[pl.ds(r0 + t * _C, _C)])
        plsc.subcore_barrier()

        @pl.when(c == 0)
        def _edges0():
            _edge_loop(hw0_h, src_h, dst_h, w_h, acc_s, rows_v, src_v,
                       dst_v, w_v, sem, s, ept16)

        @pl.when(c == 1)
        def _edges1():
            _edge_loop(hw1_h, src_h, dst_h, w_h, acc_s, rows_v, src_v,
                       dst_v, w_v, sem, s, ept16)

        plsc.subcore_barrier()

        @pl.when(c == 0)
        def _wb0():
            pltpu.sync_copy(acc_s.at[pl.ds(r0, _RPT)],
                            r0_h.at[pl.ds(r0, _RPT)])

        @pl.when(c == 1)
        def _wb1():
            pltpu.sync_copy(acc_s.at[pl.ds(r0, _RPT)],
                            r1_h.at[pl.ds(r0, _RPT)])

    return k(hw0, hw1, src, dst, w)


def _dinv(deg_block):
    return jnp.where(deg_block > 0.0, lax.rsqrt(deg_block), 0.0)


def _mm_body(prev_bias, prev_relu, h0, h1, w, d, b0, b1, o0, o1):
    dv = _dinv(d[...])
    if prev_bias:
        a0 = dv * h0[...] + b0[...]
        a1 = dv * h1[...] + b1[...]
        if prev_relu:
            a0 = jnp.maximum(a0, 0.0)
            a1 = jnp.maximum(a1, 0.0)
    else:
        a0, a1 = h0[...], h1[...]
    h = jnp.concatenate([a0, a1], axis=1)
    o0[...] = dv * jnp.dot(h, w[...][:, :_H], preferred_element_type=F32)
    o1[...] = dv * jnp.dot(h, w[...][:, _H:], preferred_element_type=F32)


def _mm(h0, h1, w, deg_col, bias=None, prev_relu=False):
    rb = 1024
    grid = (_NP // rb,)
    if bias is None:
        b0 = jnp.zeros((1, _H), F32)
        b1 = jnp.zeros((1, _H), F32)
    else:
        b0 = bias[:_H].reshape(1, _H)
        b1 = bias[_H:].reshape(1, _H)
    body = functools.partial(_mm_body, bias is not None, prev_relu)
    return pl.pallas_call(
        body,
        grid=grid,
        in_specs=[
            pl.BlockSpec((rb, _H), lambda i: (i, 0)),
            pl.BlockSpec((rb, _H), lambda i: (i, 0)),
            pl.BlockSpec((2 * _H, 2 * _H), lambda i: (0, 0)),
            pl.BlockSpec((rb, 1), lambda i: (i, 0)),
            pl.BlockSpec((1, _H), lambda i: (0, 0)),
            pl.BlockSpec((1, _H), lambda i: (0, 0)),
        ],
        out_specs=[
            pl.BlockSpec((rb, _H), lambda i: (i, 0)),
            pl.BlockSpec((rb, _H), lambda i: (i, 0)),
        ],
        out_shape=[
            jax.ShapeDtypeStruct((_NP, _H), F32),
            jax.ShapeDtypeStruct((_NP, _H), F32),
        ],
    )(h0, h1, w, deg_col, b0, b1)


def _head_body(h0, h1, d, b0, b1, batch, w0, bb0, w1, bb1, w2, bb2, out,
               sums, cnt):
    i = pl.program_id(0)
    nblk = pl.num_programs(0)

    @pl.when(i == 0)
    def _init():
        sums[...] = jnp.zeros_like(sums)
        cnt[...] = jnp.zeros_like(cnt)

    dv = _dinv(d[...])
    a0 = jnp.maximum(dv * h0[...] + b0[...], 0.0)
    a1 = jnp.maximum(dv * h1[...] + b1[...], 0.0)
    h = jnp.concatenate([a0, a1], axis=1)
    rb = h.shape[0]
    gids = lax.broadcasted_iota(I32, (rb, _G), 1)
    p = (batch[...] == gids).astype(F32)
    sums[...] += lax.dot_general(p, h, (((0,), (0,)), ((), ())),
                                 preferred_element_type=F32)
    ones = jnp.ones((rb, 1), F32)
    cnt[...] += lax.dot_general(p, ones, (((0,), (0,)), ((), ())),
                                preferred_element_type=F32)

    @pl.when(i == nblk - 1)
    def _final():
        pooled = sums[...] / jnp.maximum(cnt[...], 1.0)
        z = jnp.maximum(jnp.dot(pooled, w0[...], preferred_element_type=F32)
                        + bb0[...], 0.0)
        z = jnp.maximum(jnp.dot(z, w1[...], preferred_element_type=F32)
                        + bb1[...], 0.0)
        out[...] = jnp.dot(z, w2[...], preferred_element_type=F32) + bb2[...]


def _head(h0, h1, deg_col, bias, batch_col, w0, bb0, w1, bb1, w2, bb2):
    rb = 1024
    grid = (_NP // rb,)
    b0 = bias[:_H].reshape(1, _H)
    b1 = bias[_H:].reshape(1, _H)
    return pl.pallas_call(
        _head_body,
        grid=grid,
        in_specs=[
            pl.BlockSpec((rb, _H), lambda i: (i, 0)),
            pl.BlockSpec((rb, _H), lambda i: (i, 0)),
            pl.BlockSpec((rb, 1), lambda i: (i, 0)),
            pl.BlockSpec((1, _H), lambda i: (0, 0)),
            pl.BlockSpec((1, _H), lambda i: (0, 0)),
            pl.BlockSpec((rb, 1), lambda i: (i, 0)),
            pl.BlockSpec((2 * _H, _H), lambda i: (0, 0)),
            pl.BlockSpec((1, _H), lambda i: (0, 0)),
            pl.BlockSpec((_H, _G), lambda i: (0, 0)),
            pl.BlockSpec((1, _G), lambda i: (0, 0)),
            pl.BlockSpec((_G, _H), lambda i: (0, 0)),
            pl.BlockSpec((1, _H), lambda i: (0, 0)),
        ],
        out_specs=pl.BlockSpec((_G, _H), lambda i: (0, 0)),
        out_shape=jax.ShapeDtypeStruct((_G, _H), F32),
        scratch_shapes=[
            pltpu.VMEM((_G, 2 * _H), F32),
            pltpu.VMEM((_G, 1), F32),
        ],
    )(h0, h1, deg_col, b0, b1, batch_col, w0, bb0, w1, bb1, w2, bb2)


def kernel(x, edge_index, edge_weight, batch, W1, b1, W2, b2, W3, b3,
           Wc0, bc0, Wc1, bc1, Wc2, bc2):
    n = x.shape[0]
    e = edge_weight.shape[0]
    etot = e + n

    loop = jnp.arange(n, dtype=I32)
    src = jnp.concatenate([edge_index[0].astype(I32), loop])
    dst = jnp.concatenate([edge_index[1].astype(I32), loop])
    w = jnp.concatenate([edge_weight.astype(F32), jnp.ones((n,), F32)])

    ept16 = -(-etot // (_NT * _C)) * _C
    epad = ept16 * _NT
    pad = epad - etot
    src = jnp.pad(src, (0, pad))
    dst = jnp.pad(dst, (0, pad))
    w = jnp.pad(w, (0, pad))

    x0 = jnp.pad(x[:, :_H], ((0, _NP - n), (0, 0)))
    x1 = jnp.pad(x[:, _H:], ((0, _NP - n), (0, 0)))
    batch_col = jnp.concatenate(
        [batch.astype(I32), jnp.full((_NP - n,), _G, I32)]).reshape(_NP, 1)

    deg0, deg1 = _deg_call(dst, w, ept16 // _NC)
    deg_col = deg0[:, :1] + deg1[:, :1]

    hw0, hw1 = _mm(x0, x1, W1, deg_col)
    r10, r11 = _spmm_call(hw0, hw1, src, dst, w, ept16)
    hw0, hw1 = _mm(r10, r11, W2, deg_col, bias=b1, prev_relu=True)
    r20, r21 = _spmm_call(hw0, hw1, src, dst, w, ept16)
    hw0, hw1 = _mm(r20, r21, W3, deg_col, bias=b2, prev_relu=False)
    r30, r31 = _spmm_call(hw0, hw1, src, dst, w, ept16)

    w2p = jnp.pad(Wc2, ((0, 0), (0, _H - Wc2.shape[1])))
    b2p = jnp.pad(bc2, (0, _H - bc2.shape[0]))
    out = _head(r30, r31, deg_col, b3, batch_col,
                Wc0, bc0.reshape(1, -1), Wc1, bc1.reshape(1, -1),
                w2p, b2p.reshape(1, -1))
    return out[:, :Wc2.shape[1]]
